# baseline (speedup 1.0000x reference)
import jax
import jax.numpy as jnp
from jax import lax
from jax.experimental import pallas as pl
from jax.experimental.pallas import tpu as pltpu

B, C, H, W = 8, 192, 224, 224
HW = H * W
ROWS = B * C
C2 = 2 * C
TOPK = 64
NPLANES = B * TOPK

_RB = 128
_CB = 7168
_GR = ROWS // _RB
_GC = HW // _CB


def _pool_body(x0_ref, x1_ref, s0_ref, s1_ref):
    j = pl.program_id(1)

    @pl.when(j == 0)
    def _():
        s0_ref[...] = jnp.zeros_like(s0_ref)
        s1_ref[...] = jnp.zeros_like(s1_ref)

    s0_ref[0, 0, :] += jnp.sum(x0_ref[...], axis=1)
    s1_ref[0, 0, :] += jnp.sum(x1_ref[...], axis=1)


def _topk_body(s0_ref, s1_ref, r0_ref, r1_ref, u0_ref):
    vals = jnp.concatenate([s0_ref[...], s1_ref[...]], axis=1) / float(HW)
    iota_c = lax.broadcasted_iota(jnp.int32, (B, C2), 1)
    iota_k = lax.broadcasted_iota(jnp.int32, (B, TOPK), 1)
    idxm = jnp.zeros((B, TOPK), jnp.int32)
    for k in range(TOPK):
        m = jnp.max(vals, axis=1, keepdims=True)
        cand = jnp.where(vals == m, iota_c, jnp.int32(2**30))
        sel = jnp.min(cand, axis=1)
        idxm = jnp.where(iota_k == k, sel[:, None], idxm)
        vals = jnp.where(iota_c == sel[:, None], -jnp.inf, vals)
    rowbase = lax.broadcasted_iota(jnp.int32, (B, TOPK), 0) * C
    r0_ref[...] = rowbase + jnp.minimum(idxm, C - 1)
    r1_ref[...] = rowbase + jnp.maximum(idxm - C, 0)
    u0_ref[...] = (idxm < C).astype(jnp.int32)


_NBUF = 8


def _gather_body(r0s, r1s, u0s, x0_hbm, x1_hbm, o_hbm, sems):
    def issue(i):
        sem = sems.at[lax.rem(i, _NBUF)]

        @pl.when(u0s[i] == 1)
        def _():
            pltpu.make_async_copy(x0_hbm.at[pl.ds(r0s[i], 1)],
                                  o_hbm.at[pl.ds(i, 1)], sem).start()

        @pl.when(u0s[i] == 0)
        def _():
            pltpu.make_async_copy(x1_hbm.at[pl.ds(r1s[i], 1)],
                                  o_hbm.at[pl.ds(i, 1)], sem).start()

    def wait(i):
        pltpu.make_async_copy(x0_hbm.at[pl.ds(r0s[i], 1)],
                              o_hbm.at[pl.ds(i, 1)],
                              sems.at[lax.rem(i, _NBUF)]).wait()

    for i in range(_NBUF):
        issue(i)

    def step(i, carry):
        wait(i - _NBUF)
        issue(i)
        return carry

    lax.fori_loop(_NBUF, NPLANES, step, 0)
    for i in range(NPLANES - _NBUF, NPLANES):
        wait(i)


def kernel(x_0, x_1):
    x0r = x_0.reshape(ROWS, HW)
    x1r = x_1.reshape(ROWS, HW)

    s0, s1 = pl.pallas_call(
        _pool_body,
        grid=(_GR, _GC),
        in_specs=[pl.BlockSpec((_RB, _CB), lambda i, j: (i, j)),
                  pl.BlockSpec((_RB, _CB), lambda i, j: (i, j))],
        out_specs=[pl.BlockSpec((1, 1, _RB), lambda i, j: (i, 0, 0)),
                   pl.BlockSpec((1, 1, _RB), lambda i, j: (i, 0, 0))],
        out_shape=[jax.ShapeDtypeStruct((_GR, 1, _RB), jnp.float32),
                   jax.ShapeDtypeStruct((_GR, 1, _RB), jnp.float32)],
        compiler_params=pltpu.CompilerParams(
            dimension_semantics=("parallel", "arbitrary")),
    )(x0r, x1r)

    r0, r1, u0 = pl.pallas_call(
        _topk_body,
        out_shape=[jax.ShapeDtypeStruct((B, TOPK), jnp.int32)] * 3,
    )(s0.reshape(B, C), s1.reshape(B, C))

    out = pl.pallas_call(
        _gather_body,
        in_specs=[
            pl.BlockSpec(memory_space=pltpu.SMEM),
            pl.BlockSpec(memory_space=pltpu.SMEM),
            pl.BlockSpec(memory_space=pltpu.SMEM),
            pl.BlockSpec(memory_space=pl.ANY),
            pl.BlockSpec(memory_space=pl.ANY),
        ],
        out_specs=pl.BlockSpec(memory_space=pl.ANY),
        out_shape=jax.ShapeDtypeStruct((NPLANES, HW), jnp.float32),
        scratch_shapes=[pltpu.SemaphoreType.DMA((_NBUF,))],
    )(r0.reshape(-1), r1.reshape(-1), u0.reshape(-1), x0r, x1r)

    return out.reshape(B, TOPK, H, W)


# speedup vs baseline: 3.8378x; 3.8378x over previous
import functools

import jax
import jax.numpy as jnp
from jax import lax
from jax.experimental import pallas as pl
from jax.experimental.pallas import tpu as pltpu
from jax.experimental.pallas import tpu_sc as plsc

B, C, H, W = 8, 192, 224, 224
HW = H * W
ROWS = B * C
C2 = 2 * C
TOPK = 64
NPLANES = B * TOPK

_RB = 128
_CB = 7168
_GR = ROWS // _RB
_GC = HW // _CB


def _pool_body(x0_ref, x1_ref, s0_ref, s1_ref):
    j = pl.program_id(1)

    @pl.when(j == 0)
    def _():
        s0_ref[...] = jnp.zeros_like(s0_ref)
        s1_ref[...] = jnp.zeros_like(s1_ref)

    s0_ref[0, 0, :] += jnp.sum(x0_ref[...], axis=1)
    s1_ref[0, 0, :] += jnp.sum(x1_ref[...], axis=1)


def _topk_body(s0_ref, s1_ref, r0_ref, r1_ref, u0_ref):
    vals = jnp.concatenate([s0_ref[...], s1_ref[...]], axis=1) / float(HW)
    iota_c = lax.broadcasted_iota(jnp.int32, (B, C2), 1)
    iota_k = lax.broadcasted_iota(jnp.int32, (B, TOPK), 1)
    idxm = jnp.zeros((B, TOPK), jnp.int32)
    for k in range(TOPK):
        m = jnp.max(vals, axis=1, keepdims=True)
        cand = jnp.where(vals == m, iota_c, jnp.int32(2**30))
        sel = jnp.min(cand, axis=1)
        idxm = jnp.where(iota_k == k, sel[:, None], idxm)
        vals = jnp.where(iota_c == sel[:, None], -jnp.inf, vals)
    rowbase = lax.broadcasted_iota(jnp.int32, (B, TOPK), 0) * C
    r0_ref[...] = rowbase + jnp.minimum(idxm, C - 1)
    r1_ref[...] = rowbase + jnp.maximum(idxm - C, 0)
    u0_ref[...] = (idxm < C).astype(jnp.int32)


_NW = 32
_PPW = NPLANES // _NW


def _sc_gather_body(r0_hbm, r1_hbm, u0_hbm, x0_hbm, x1_hbm, o_hbm,
                    idx_v, b0, b1, g0, g1, s0, s1):
    cid = lax.axis_index("c")
    sid = lax.axis_index("s")
    wid = sid * 2 + cid
    base = wid * _PPW
    pltpu.sync_copy(r0_hbm.at[pl.ds(base, _PPW)], idx_v.at[0])
    pltpu.sync_copy(r1_hbm.at[pl.ds(base, _PPW)], idx_v.at[1])
    pltpu.sync_copy(u0_hbm.at[pl.ds(base, _PPW)], idx_v.at[2])
    bufs = (b0, b1)
    gsems = (g0, g1)
    ssems = (s0, s1)

    r0v = idx_v[0]
    r1v = idx_v[1]
    u0v = idx_v[2]
    rv = jnp.where(u0v == 1, r0v, r1v)

    def row(j):
        return rv[j]

    def issue_gather(j):
        r = row(j)
        u = u0v[j]
        buf, sem = bufs[j % 2], gsems[j % 2]

        @pl.when(u == 1)
        def _():
            pltpu.make_async_copy(x0_hbm.at[pl.ds(r, 1)], buf, sem).start()

        @pl.when(u == 0)
        def _():
            pltpu.make_async_copy(x1_hbm.at[pl.ds(r, 1)], buf, sem).start()

    def wait_gather(j):
        pltpu.make_async_copy(x0_hbm.at[pl.ds(row(j), 1)], bufs[j % 2],
                              gsems[j % 2]).wait()

    def issue_scatter(j):
        pltpu.make_async_copy(bufs[j % 2], o_hbm.at[pl.ds(base + j, 1)],
                              ssems[j % 2]).start()

    def wait_scatter(j):
        pltpu.make_async_copy(bufs[j % 2], o_hbm.at[pl.ds(base + j, 1)],
                              ssems[j % 2]).wait()

    issue_gather(0)
    issue_gather(1)
    for j in range(_PPW):
        wait_gather(j)
        issue_scatter(j)
        if j + 2 < _PPW:
            wait_scatter(j)
            issue_gather(j + 2)
    wait_scatter(_PPW - 2)
    wait_scatter(_PPW - 1)


def kernel(x_0, x_1):
    x0r = x_0.reshape(ROWS, HW)
    x1r = x_1.reshape(ROWS, HW)

    s0, s1 = pl.pallas_call(
        _pool_body,
        grid=(_GR, _GC),
        in_specs=[pl.BlockSpec((_RB, _CB), lambda i, j: (i, j)),
                  pl.BlockSpec((_RB, _CB), lambda i, j: (i, j))],
        out_specs=[pl.BlockSpec((1, 1, _RB), lambda i, j: (i, 0, 0)),
                   pl.BlockSpec((1, 1, _RB), lambda i, j: (i, 0, 0))],
        out_shape=[jax.ShapeDtypeStruct((_GR, 1, _RB), jnp.float32),
                   jax.ShapeDtypeStruct((_GR, 1, _RB), jnp.float32)],
        compiler_params=pltpu.CompilerParams(
            dimension_semantics=("parallel", "arbitrary")),
    )(x0r, x1r)

    r0, r1, u0 = pl.pallas_call(
        _topk_body,
        out_shape=[jax.ShapeDtypeStruct((B, TOPK), jnp.int32)] * 3,
    )(s0.reshape(B, C), s1.reshape(B, C))

    sc_gather = functools.partial(
        pl.kernel,
        mesh=plsc.VectorSubcoreMesh(core_axis_name="c", subcore_axis_name="s"),
        out_type=jax.ShapeDtypeStruct((NPLANES, HW), jnp.float32),
        scratch_types=[
            pltpu.VMEM((3, _PPW), jnp.int32),
            pltpu.VMEM((1, HW), jnp.float32),
            pltpu.VMEM((1, HW), jnp.float32),
            pltpu.SemaphoreType.DMA,
            pltpu.SemaphoreType.DMA,
            pltpu.SemaphoreType.DMA,
            pltpu.SemaphoreType.DMA,
        ],
    )(_sc_gather_body)
    out = sc_gather(r0.reshape(-1), r1.reshape(-1), u0.reshape(-1), x0r, x1r)

    return out.reshape(B, TOPK, H, W)
